# baseline (speedup 1.0000x reference)
import functools

import jax
import jax.numpy as jnp
from jax.experimental import pallas as pl
from jax.experimental.pallas import tpu as pltpu

_SR = 0.2
_FLIP = 0x7FFFFFFF


def _key_to_f32(key):
    return jax.lax.bitcast_convert_type(
        key ^ ((key >> 31) & jnp.int32(_FLIP)), jnp.float32)


def _kwta_block(x_ref, o_ref, keys_ref, *, k: int, w: int):
    rows, n = x_ref.shape
    chunks = n // w
    i32min = jnp.iinfo(jnp.int32).min
    i32max = jnp.iinfo(jnp.int32).max

    mn = jnp.full((rows, w), i32max, dtype=jnp.int32)
    mx = jnp.full((rows, w), i32min, dtype=jnp.int32)
    for j in range(chunks):
        sl = slice(j * w, (j + 1) * w)
        b = jax.lax.bitcast_convert_type(x_ref[:, sl], jnp.int32)
        s = b ^ ((b >> 31) & jnp.int32(_FLIP))
        keys_ref[:, sl] = s
        mn = jnp.minimum(mn, s)
        mx = jnp.maximum(mx, s)
    lo = jnp.min(mn, axis=1, keepdims=True)
    hi = jnp.max(mx, axis=1, keepdims=True)
    clo = jnp.full((rows, 1), n, dtype=jnp.int32)
    chi = jnp.zeros((rows, 1), dtype=jnp.int32)

    def count_ge(mid):
        acc = jnp.zeros((rows, w), dtype=jnp.int32)
        for j in range(chunks):
            acc = acc + jnp.where(keys_ref[:, j * w:(j + 1) * w] >= mid, 1, 0)
        return jnp.sum(acc, axis=1, keepdims=True)

    def cond(carry):
        _, lo, hi, clo, _ = carry
        return jnp.any(jnp.logical_and(hi > lo, clo != k))

    def body(carry):
        it, lo, hi, clo, chi = carry
        done = jnp.logical_or(hi <= lo, clo == k)
        mid_bis = (lo >> 1) + (hi >> 1) + ((lo | hi) & 1)
        vlo = _key_to_f32(lo)
        vhi = _key_to_f32(hi)
        frac = (clo - k).astype(jnp.float32) / jnp.maximum(
            (clo - chi).astype(jnp.float32), 1.0)
        tv = vlo + frac * (vhi - vlo)
        tb = jax.lax.bitcast_convert_type(tv, jnp.int32)
        tk = tb ^ ((tb >> 31) & jnp.int32(_FLIP))
        mid_interp = jnp.clip(tk, lo + 1, hi)
        mid = jnp.where((it & 7) == 7, mid_bis, mid_interp)
        mid = jnp.where(done, lo, mid)
        cnt = count_ge(mid)
        ge = cnt >= k
        upd = jnp.logical_not(done)
        lo = jnp.where(jnp.logical_and(upd, ge), mid, lo)
        clo = jnp.where(jnp.logical_and(upd, ge), cnt, clo)
        hi = jnp.where(jnp.logical_and(upd, jnp.logical_not(ge)), mid - 1, hi)
        chi = jnp.where(jnp.logical_and(upd, jnp.logical_not(ge)), cnt, chi)
        return it + 1, lo, hi, clo, chi

    carry = (jnp.int32(0), lo, hi, clo, chi)
    carry = jax.lax.fori_loop(0, 8, lambda i, c: body(c), carry)
    _, lo, hi, clo, chi = jax.lax.while_loop(
        cond, lambda c: body(body(c)), carry)

    tm = jnp.full((rows, w), i32max, dtype=jnp.int32)
    for j in range(chunks):
        s = keys_ref[:, j * w:(j + 1) * w]
        tm = jnp.minimum(tm, jnp.where(s >= lo, s, i32max))
    tmin = jnp.min(tm, axis=1, keepdims=True)
    thr = jnp.where(clo == k, tmin, lo)

    xb = x_ref[...]
    o_ref[...] = jnp.where(xb >= _key_to_f32(thr), xb, jnp.float32(0.0))


def kernel(x):
    B, N = x.shape
    k = int(_SR * N)
    block_rows = 64
    grid = (B // block_rows,)
    return pl.pallas_call(
        functools.partial(_kwta_block, k=k, w=256),
        grid=grid,
        in_specs=[pl.BlockSpec((block_rows, N), lambda i: (i, 0))],
        out_specs=pl.BlockSpec((block_rows, N), lambda i: (i, 0)),
        out_shape=jax.ShapeDtypeStruct((B, N), x.dtype),
        scratch_shapes=[pltpu.VMEM((block_rows, N), jnp.int32)],
        compiler_params=pltpu.CompilerParams(
            dimension_semantics=("arbitrary",)),
    )(x)


# speedup vs baseline: 1.1973x; 1.1973x over previous
import functools

import jax
import jax.numpy as jnp
from jax.experimental import pallas as pl
from jax.experimental.pallas import tpu as pltpu

_SR = 0.2
_FLIP = 0x7FFFFFFF


def _key_to_f32(key):
    return jax.lax.bitcast_convert_type(
        key ^ ((key >> 31) & jnp.int32(_FLIP)), jnp.float32)


def _kwta_block(x_ref, o_ref, keys_ref, *, k: int, w: int):
    rows, n = x_ref.shape
    chunks = n // w
    i32min = jnp.iinfo(jnp.int32).min
    i32max = jnp.iinfo(jnp.int32).max

    mn = jnp.full((rows, w), i32max, dtype=jnp.int32)
    mx = jnp.full((rows, w), i32min, dtype=jnp.int32)
    s1 = jnp.zeros((rows, w), dtype=jnp.float32)
    s2 = jnp.zeros((rows, w), dtype=jnp.float32)
    for j in range(chunks):
        sl = slice(j * w, (j + 1) * w)
        xs = x_ref[:, sl]
        b = jax.lax.bitcast_convert_type(xs, jnp.int32)
        s = b ^ ((b >> 31) & jnp.int32(_FLIP))
        keys_ref[:, sl] = s
        mn = jnp.minimum(mn, s)
        mx = jnp.maximum(mx, s)
        s1 = s1 + xs
        s2 = s2 + xs * xs
    lo = jnp.min(mn, axis=1, keepdims=True)
    hi = jnp.max(mx, axis=1, keepdims=True)
    mu = jnp.sum(s1, axis=1, keepdims=True) / n
    ex2 = jnp.sum(s2, axis=1, keepdims=True) / n
    sd = jnp.sqrt(jnp.maximum(ex2 - mu * mu, 0.0))
    zq = jnp.float32(0.8416212)
    q0 = mu + zq * sd
    clo = jnp.full((rows, 1), n, dtype=jnp.int32)
    chi = jnp.zeros((rows, 1), dtype=jnp.int32)

    def count_ge(mid):
        acc = jnp.zeros((rows, w), dtype=jnp.int32)
        for j in range(chunks):
            acc = acc + jnp.where(keys_ref[:, j * w:(j + 1) * w] >= mid, 1, 0)
        return jnp.sum(acc, axis=1, keepdims=True)

    def cond(carry):
        _, lo, hi, clo, _ = carry
        return jnp.any(jnp.logical_and(hi > lo, clo != k))

    def body(carry):
        it, lo, hi, clo, chi = carry
        done = jnp.logical_or(hi <= lo, clo == k)
        mid_bis = (lo >> 1) + (hi >> 1) + ((lo | hi) & 1)
        vlo = _key_to_f32(lo)
        vhi = _key_to_f32(hi)
        frac = (clo - k).astype(jnp.float32) / jnp.maximum(
            (clo - chi).astype(jnp.float32), 1.0)
        tv = vlo + frac * (vhi - vlo)
        tv = jnp.where(it == 0, q0, tv)
        tb = jax.lax.bitcast_convert_type(tv, jnp.int32)
        tk = tb ^ ((tb >> 31) & jnp.int32(_FLIP))
        mid_interp = jnp.clip(tk, lo + 1, hi)
        mid = jnp.where((it & 7) == 7, mid_bis, mid_interp)
        mid = jnp.where(done, lo, mid)
        cnt = count_ge(mid)
        ge = cnt >= k
        upd = jnp.logical_not(done)
        lo = jnp.where(jnp.logical_and(upd, ge), mid, lo)
        clo = jnp.where(jnp.logical_and(upd, ge), cnt, clo)
        hi = jnp.where(jnp.logical_and(upd, jnp.logical_not(ge)), mid - 1, hi)
        chi = jnp.where(jnp.logical_and(upd, jnp.logical_not(ge)), cnt, chi)
        return it + 1, lo, hi, clo, chi

    carry = (jnp.int32(0), lo, hi, clo, chi)
    carry = jax.lax.fori_loop(0, 6, lambda i, c: body(c), carry)
    _, lo, hi, clo, chi = jax.lax.while_loop(cond, body, carry)

    tm = jnp.full((rows, w), i32max, dtype=jnp.int32)
    for j in range(chunks):
        s = keys_ref[:, j * w:(j + 1) * w]
        tm = jnp.minimum(tm, jnp.where(s >= lo, s, i32max))
    tmin = jnp.min(tm, axis=1, keepdims=True)
    thr = jnp.where(clo == k, tmin, lo)

    xb = x_ref[...]
    o_ref[...] = jnp.where(xb >= _key_to_f32(thr), xb, jnp.float32(0.0))


def kernel(x):
    B, N = x.shape
    k = int(_SR * N)
    block_rows = 64
    grid = (B // block_rows,)
    return pl.pallas_call(
        functools.partial(_kwta_block, k=k, w=256),
        grid=grid,
        in_specs=[pl.BlockSpec((block_rows, N), lambda i: (i, 0))],
        out_specs=pl.BlockSpec((block_rows, N), lambda i: (i, 0)),
        out_shape=jax.ShapeDtypeStruct((B, N), x.dtype),
        scratch_shapes=[pltpu.VMEM((block_rows, N), jnp.int32)],
        compiler_params=pltpu.CompilerParams(
            dimension_semantics=("arbitrary",)),
    )(x)
